# Initial kernel scaffold; baseline (speedup 1.0000x reference)
import jax
import jax.numpy as jnp
from jax.experimental import pallas as pl


def kernel(nfeat, edge_index, efeat, fc_w, attn_l, attn_r, gat_bias, edge_w, edge_b):
    raise NotImplementedError("write your pallas kernel here")


# speedup vs baseline: 33.8928x; 33.8928x over previous
import functools

import jax
import jax.numpy as jnp
from jax import lax
from jax.experimental import pallas as pl
from jax.experimental.pallas import tpu as pltpu
from jax.experimental.pallas import tpu_sc as plsc

N = 10000
E = 320000
D = 128
H = 8
OUT = 16
NC, NS, L = 2, 16, 16
C = 128
NCHUNK = E // C
NP_ = 10112
ROWS_PER_TILE = NP_ // NS
DDR = 704
DDC = 32
DDZ = 48
QC = C // 4

_BLK = 1000
_GRID = N // _BLK


def _proj_body(x_ref, w_ref, alr_ref, feat_ref, elr_ref):
    f = jnp.dot(x_ref[...], w_ref[...], preferred_element_type=jnp.float32)
    feat_ref[...] = f
    elr_ref[...] = jnp.dot(f, alr_ref[...], preferred_element_type=jnp.float32)


def _project(nfeat, w1, alr):
    return pl.pallas_call(
        _proj_body,
        grid=(_GRID,),
        in_specs=[
            pl.BlockSpec((_BLK, D), lambda i: (i, 0)),
            pl.BlockSpec((D, D), lambda i: (0, 0)),
            pl.BlockSpec((D, L), lambda i: (0, 0)),
        ],
        out_specs=[
            pl.BlockSpec((_BLK, D), lambda i: (i, 0)),
            pl.BlockSpec((_BLK, L), lambda i: (i, 0)),
        ],
        out_shape=[
            jax.ShapeDtypeStruct((N, D), jnp.float32),
            jax.ShapeDtypeStruct((N, L), jnp.float32),
        ],
    )(nfeat, w1, alr)


_mesh = plsc.VectorSubcoreMesh(
    core_axis_name="c", subcore_axis_name="s", num_cores=NC, num_subcores=NS)


@functools.partial(
    pl.kernel,
    out_type=(
        jax.ShapeDtypeStruct((NC, NP_, D), jnp.float32),
        jax.ShapeDtypeStruct((NP_, L), jnp.float32),
    ),
    mesh=_mesh,
    scratch_types=[
        pltpu.VMEM_SHARED((NP_, D), jnp.float32),
        pltpu.VMEM_SHARED((NP_, L), jnp.float32),
        pltpu.VMEM_SHARED((N, L), jnp.float32),
        pltpu.VMEM((2, C), jnp.int32),
        pltpu.VMEM((2, QC, L), jnp.float32),
        pltpu.VMEM((C, D), jnp.float32),
        pltpu.VMEM((C, L), jnp.float32),
        pltpu.SemaphoreType.DMA,
        pltpu.SemaphoreType.DMA,
    ],
    compiler_params=pltpu.CompilerParams(needs_layout_passes=False,
                                         use_tc_tiling_on_sc=False),
)
def _edge_kernel(src_hbm, dst_hbm, elr_hbm, feat_hbm, efeat_hbm,
                 acc_out, dd_out,
                 acc_sh, dd_sh, elr_sh,
                 idx2, eb2, featb, wsc,
                 sem0, sem2):
    cid = lax.axis_index("c")
    sid = lax.axis_index("s")
    lanes = lax.iota(jnp.int32, L)
    lane_lt8 = lanes < 8
    lane8_one = jnp.where(lanes == 8, 1.0, 0.0).astype(jnp.float32)

    def _zrow(i, carry):
        for j in range(D // L):
            featb[i, pl.ds(j * L, L)] = jnp.zeros((L,), jnp.float32)
        wsc[i, :] = jnp.zeros((L,), jnp.float32)
        return carry
    lax.fori_loop(0, C, _zrow, 0)

    base = sid * ROWS_PER_TILE
    for k in range(0, ROWS_PER_TILE, C):
        cnt = min(C, ROWS_PER_TILE - k)
        pltpu.sync_copy(featb.at[pl.ds(0, cnt)], acc_sh.at[pl.ds(base + k, cnt)])
        pltpu.sync_copy(wsc.at[pl.ds(0, cnt)], dd_sh.at[pl.ds(base + k, cnt)])

    @pl.when(cid == 0)
    def _stage():
        sbase = jnp.minimum(sid * 640, N - 640)
        pltpu.sync_copy(elr_hbm.at[pl.ds(sbase, 640)],
                        elr_sh.at[pl.ds(sbase, 640)])

    plsc.subcore_barrier()

    nloops = (NCHUNK + NS - 1) // NS

    @pl.when(cid == 0)
    def _attention_core():
        def _chunk(j, carry):
            ck = sid + j * NS

            @pl.when(ck < NCHUNK)
            def _():
                eb = ck * C
                pltpu.sync_copy(src_hbm.at[pl.ds(eb, C)], idx2.at[0])
                pltpu.sync_copy(dst_hbm.at[pl.ds(eb, C)], idx2.at[1])
                g3 = pltpu.async_copy(feat_hbm.at[idx2.at[0]], featb, sem2)
                g3.wait()

                def _half(half, c2):
                    hb = half * QC

                    def _gat(t, c3):
                        pltpu.async_copy(
                            elr_sh.at[idx2.at[t, pl.ds(hb, QC)]],
                            eb2.at[t], sem0).wait()
                        return c3
                    lax.fori_loop(0, 2, _gat, 0)

                    def _erow(i, c3):
                        rot = eb2[1, i, :].at[(lanes + 8) % L].get(
                            mode="promise_in_bounds")
                        x = eb2[0, i, :] + rot
                        w = jnp.exp(jnp.maximum(x, 0.2 * x))
                        wsc[hb + i, :] = jnp.where(lane_lt8, w, lane8_one)
                        for h in range(H):
                            wh = w.at[jnp.full((L,), h, jnp.int32)].get(
                                mode="promise_in_bounds")
                            featb[hb + i, pl.ds(h * L, L)] = (
                                featb[hb + i, pl.ds(h * L, L)] * wh)
                        return c3
                    lax.fori_loop(0, QC, _erow, 0)
                    return c2
                lax.fori_loop(0, 4, _half, 0)

                pltpu.sync_copy(featb, acc_sh.at[idx2.at[1]], add=True)
                pltpu.sync_copy(wsc, dd_sh.at[idx2.at[1]], add=True)
            return carry
        lax.fori_loop(0, nloops, _chunk, 0)

    @pl.when(cid == 1)
    def _esum_core():
        def _chunk(j, carry):
            ck = sid + j * NS

            @pl.when(ck < NCHUNK)
            def _():
                eb = ck * C
                pltpu.sync_copy(dst_hbm.at[pl.ds(eb, C)], idx2.at[1])
                pltpu.sync_copy(efeat_hbm.at[pl.ds(eb, C)], featb)
                pltpu.sync_copy(featb, acc_sh.at[idx2.at[1]], add=True)
            return carry
        lax.fori_loop(0, nloops, _chunk, 0)

    plsc.subcore_barrier()

    pltpu.sync_copy(acc_sh.at[pl.ds(base, ROWS_PER_TILE)],
                    acc_out.at[cid, pl.ds(base, ROWS_PER_TILE)])

    @pl.when(cid == 0)
    def _drain_dd():
        pltpu.sync_copy(dd_sh.at[pl.ds(base, ROWS_PER_TILE)],
                        dd_out.at[pl.ds(base, ROWS_PER_TILE)])


def _comb_body(num_ref, esum_ref, dd_ref, deg_sel_ref, feat_ref, ew_ref, e16_ref,
               gb_ref, eb_ref, o_ref):
    dd = dd_ref[...]
    denom = jnp.dot(dd, e16_ref[...], preferred_element_type=jnp.float32)
    deg = jnp.dot(dd, deg_sel_ref[...], preferred_element_type=jnp.float32)
    rst = num_ref[...] / jnp.maximum(denom, 1e-9)
    he = (jnp.dot(esum_ref[...], ew_ref[...], preferred_element_type=jnp.float32)
          / jnp.maximum(deg, 1.0)) + eb_ref[...] * jnp.minimum(deg, 1.0)
    o_ref[...] = rst + gb_ref[...] + feat_ref[...] / (deg + 1.0) + he


def _combine(num, esum, dn, deg, feat, ewT, e8, gb, eb):
    return pl.pallas_call(
        _comb_body,
        grid=(_GRID,),
        in_specs=[
            pl.BlockSpec((_BLK, D), lambda i: (i, 0)),
            pl.BlockSpec((_BLK, D), lambda i: (i, 0)),
            pl.BlockSpec((_BLK, L), lambda i: (i, 0)),
            pl.BlockSpec((L, 1), lambda i: (0, 0)),
            pl.BlockSpec((_BLK, D), lambda i: (i, 0)),
            pl.BlockSpec((D, D), lambda i: (0, 0)),
            pl.BlockSpec((L, D), lambda i: (0, 0)),
            pl.BlockSpec((1, D), lambda i: (0, 0)),
            pl.BlockSpec((1, D), lambda i: (0, 0)),
        ],
        out_specs=pl.BlockSpec((_BLK, D), lambda i: (i, 0)),
        out_shape=jax.ShapeDtypeStruct((N, D), jnp.float32),
    )(num, esum, dn, deg, feat, ewT, e8, gb, eb)


def kernel(nfeat, edge_index, efeat, fc_w, attn_l, attn_r, gat_bias, edge_w,
           edge_b):
    src = edge_index[0]
    dst = edge_index[1]
    cols = jnp.arange(D)
    head = cols // OUT
    alr = (jnp.zeros((D, L), jnp.float32)
           .at[cols, head].set(attn_l.reshape(-1))
           .at[cols, 8 + head].set(attn_r.reshape(-1)))
    feat, elrtab = _project(nfeat, fc_w.T, alr)

    acc, dd = _edge_kernel(src, dst, elrtab, feat, efeat)

    e16 = (jnp.arange(L)[:, None] == head[None, :]).astype(jnp.float32)
    dsel = (jnp.arange(L)[:, None] == 8).astype(jnp.float32)
    out = _combine(acc[0, :N], acc[1, :N], dd[:N], dsel, feat, edge_w.T, e16,
                   gat_bias.reshape(1, D), edge_b.reshape(1, D))
    return out


# speedup vs baseline: 41.8260x; 1.2341x over previous
import functools

import jax
import jax.numpy as jnp
from jax import lax
from jax.experimental import pallas as pl
from jax.experimental.pallas import tpu as pltpu
from jax.experimental.pallas import tpu_sc as plsc

N = 10000
E = 320000
D = 128
H = 8
OUT = 16
NC, NS, L = 2, 16, 16
C = 128
NCHUNK = E // C
NP_ = 10112
ROWS_PER_TILE = NP_ // NS
DDR = 704
DDC = 32
DDZ = 48
QC = C // 4

_BLK = 1000
_GRID = N // _BLK


def _proj_body(x_ref, w_ref, alr_ref, feat_ref, elr_ref):
    f = jnp.dot(x_ref[...], w_ref[...], preferred_element_type=jnp.float32)
    feat_ref[...] = f
    elr_ref[...] = jnp.dot(f, alr_ref[...], preferred_element_type=jnp.float32)


def _project(nfeat, w1, alr):
    return pl.pallas_call(
        _proj_body,
        grid=(_GRID,),
        in_specs=[
            pl.BlockSpec((_BLK, D), lambda i: (i, 0)),
            pl.BlockSpec((D, D), lambda i: (0, 0)),
            pl.BlockSpec((D, L), lambda i: (0, 0)),
        ],
        out_specs=[
            pl.BlockSpec((_BLK, D), lambda i: (i, 0)),
            pl.BlockSpec((_BLK, L), lambda i: (i, 0)),
        ],
        out_shape=[
            jax.ShapeDtypeStruct((N, D), jnp.float32),
            jax.ShapeDtypeStruct((N, L), jnp.float32),
        ],
    )(nfeat, w1, alr)


_mesh = plsc.VectorSubcoreMesh(
    core_axis_name="c", subcore_axis_name="s", num_cores=NC, num_subcores=NS)


@functools.partial(
    pl.kernel,
    out_type=(
        jax.ShapeDtypeStruct((NC, NP_, D), jnp.float32),
        jax.ShapeDtypeStruct((NP_, L), jnp.float32),
    ),
    mesh=_mesh,
    scratch_types=[
        pltpu.VMEM_SHARED((NP_, D), jnp.float32),
        pltpu.VMEM_SHARED((NP_, L), jnp.float32),
        pltpu.VMEM_SHARED((N, L), jnp.float32),
        pltpu.VMEM((2, C), jnp.int32),
        pltpu.VMEM((2, C, L), jnp.float32),
        pltpu.VMEM((C, D), jnp.float32),
        pltpu.VMEM((C, L), jnp.float32),
        pltpu.SemaphoreType.DMA,
        pltpu.SemaphoreType.DMA,
        pltpu.SemaphoreType.DMA,
    ],
    compiler_params=pltpu.CompilerParams(needs_layout_passes=False,
                                         use_tc_tiling_on_sc=False),
)
def _edge_kernel(eidx_hbm, elr_hbm, feat_hbm, efeat_hbm,
                 acc_out, dd_out,
                 acc_sh, dd_sh, elr_sh,
                 idx2, eb2, featb, wsc,
                 sem0, sem1, sem2):
    cid = lax.axis_index("c")
    sid = lax.axis_index("s")
    lanes = lax.iota(jnp.int32, L)
    lane_lt8 = lanes < 8
    lane8_one = jnp.where(lanes == 8, 1.0, 0.0).astype(jnp.float32)

    def _zrow(i, carry):
        for j in range(D // L):
            featb[i, pl.ds(j * L, L)] = jnp.zeros((L,), jnp.float32)
        wsc[i, :] = jnp.zeros((L,), jnp.float32)
        return carry
    lax.fori_loop(0, C, _zrow, 0)

    base = sid * ROWS_PER_TILE
    for k in range(0, ROWS_PER_TILE, C):
        cnt = min(C, ROWS_PER_TILE - k)
        pltpu.sync_copy(featb.at[pl.ds(0, cnt)], acc_sh.at[pl.ds(base + k, cnt)])
        pltpu.sync_copy(wsc.at[pl.ds(0, cnt)], dd_sh.at[pl.ds(base + k, cnt)])

    @pl.when(cid == 0)
    def _stage():
        sbase = jnp.minimum(sid * 640, N - 640)
        pltpu.sync_copy(elr_hbm.at[pl.ds(sbase, 640)],
                        elr_sh.at[pl.ds(sbase, 640)])

    plsc.subcore_barrier()

    nloops = (NCHUNK + NS - 1) // NS

    @pl.when(cid == 0)
    def _attention_core():
        def _chunk(j, carry):
            ck = sid + j * NS

            @pl.when(ck < NCHUNK)
            def _():
                eb = ck * C
                pltpu.sync_copy(eidx_hbm.at[:, pl.ds(eb, C)], idx2)
                g3 = pltpu.async_copy(feat_hbm.at[idx2.at[0]], featb, sem2)
                g1 = pltpu.async_copy(elr_sh.at[idx2.at[0]], eb2.at[0], sem0)
                g2 = pltpu.async_copy(elr_sh.at[idx2.at[1]], eb2.at[1], sem1)
                g1.wait()
                g2.wait()
                g3.wait()

                def _erow(i, c3):
                    rot = eb2[1, i, :].at[(lanes + 8) % L].get(
                        mode="promise_in_bounds")
                    x = eb2[0, i, :] + rot
                    w = jnp.exp(jnp.maximum(x, 0.2 * x))
                    wsc[i, :] = jnp.where(lane_lt8, w, lane8_one)
                    for h in range(H):
                        wh = w.at[jnp.full((L,), h, jnp.int32)].get(
                            mode="promise_in_bounds")
                        featb[i, pl.ds(h * L, L)] = (
                            featb[i, pl.ds(h * L, L)] * wh)
                    return c3
                lax.fori_loop(0, C, _erow, 0)

                pltpu.sync_copy(featb, acc_sh.at[idx2.at[1]], add=True)
                pltpu.sync_copy(wsc, dd_sh.at[idx2.at[1]], add=True)
            return carry
        lax.fori_loop(0, nloops, _chunk, 0)

    @pl.when(cid == 1)
    def _esum_core():
        def _chunk(j, carry):
            ck = sid + j * NS

            @pl.when(ck < NCHUNK)
            def _():
                eb = ck * C
                pltpu.sync_copy(eidx_hbm.at[:, pl.ds(eb, C)], idx2)
                pltpu.sync_copy(efeat_hbm.at[pl.ds(eb, C)], featb)
                pltpu.sync_copy(featb, acc_sh.at[idx2.at[1]], add=True)
            return carry
        lax.fori_loop(0, nloops, _chunk, 0)

    plsc.subcore_barrier()

    pltpu.sync_copy(acc_sh.at[pl.ds(base, ROWS_PER_TILE)],
                    acc_out.at[cid, pl.ds(base, ROWS_PER_TILE)])

    @pl.when(cid == 0)
    def _drain_dd():
        pltpu.sync_copy(dd_sh.at[pl.ds(base, ROWS_PER_TILE)],
                        dd_out.at[pl.ds(base, ROWS_PER_TILE)])


def _comb_body(num_ref, esum_ref, dd_ref, deg_sel_ref, feat_ref, ew_ref, e16_ref,
               gb_ref, eb_ref, o_ref):
    dd = dd_ref[...]
    denom = jnp.dot(dd, e16_ref[...], preferred_element_type=jnp.float32)
    deg = jnp.dot(dd, deg_sel_ref[...], preferred_element_type=jnp.float32)
    rst = num_ref[...] / jnp.maximum(denom, 1e-9)
    he = (jnp.dot(esum_ref[...], ew_ref[...], preferred_element_type=jnp.float32)
          / jnp.maximum(deg, 1.0)) + eb_ref[...] * jnp.minimum(deg, 1.0)
    o_ref[...] = rst + gb_ref[...] + feat_ref[...] / (deg + 1.0) + he


def _combine(num, esum, dn, deg, feat, ewT, e8, gb, eb):
    return pl.pallas_call(
        _comb_body,
        grid=(_GRID,),
        in_specs=[
            pl.BlockSpec((_BLK, D), lambda i: (i, 0)),
            pl.BlockSpec((_BLK, D), lambda i: (i, 0)),
            pl.BlockSpec((_BLK, L), lambda i: (i, 0)),
            pl.BlockSpec((L, 1), lambda i: (0, 0)),
            pl.BlockSpec((_BLK, D), lambda i: (i, 0)),
            pl.BlockSpec((D, D), lambda i: (0, 0)),
            pl.BlockSpec((L, D), lambda i: (0, 0)),
            pl.BlockSpec((1, D), lambda i: (0, 0)),
            pl.BlockSpec((1, D), lambda i: (0, 0)),
        ],
        out_specs=pl.BlockSpec((_BLK, D), lambda i: (i, 0)),
        out_shape=jax.ShapeDtypeStruct((N, D), jnp.float32),
    )(num, esum, dn, deg, feat, ewT, e8, gb, eb)


def kernel(nfeat, edge_index, efeat, fc_w, attn_l, attn_r, gat_bias, edge_w,
           edge_b):
    cols = jnp.arange(D)
    head = cols // OUT
    alr = (jnp.zeros((D, L), jnp.float32)
           .at[cols, head].set(attn_l.reshape(-1))
           .at[cols, 8 + head].set(attn_r.reshape(-1)))
    feat, elrtab = _project(nfeat, fc_w.T, alr)

    acc, dd = _edge_kernel(edge_index, elrtab, feat, efeat)

    e16 = (jnp.arange(L)[:, None] == head[None, :]).astype(jnp.float32)
    dsel = (jnp.arange(L)[:, None] == 8).astype(jnp.float32)
    out = _combine(acc[0, :N], acc[1, :N], dd[:N], dsel, feat, edge_w.T, e16,
                   gat_bias.reshape(1, D), edge_b.reshape(1, D))
    return out


# speedup vs baseline: 43.5455x; 1.0411x over previous
import functools

import jax
import jax.numpy as jnp
from jax import lax
from jax.experimental import pallas as pl
from jax.experimental.pallas import tpu as pltpu
from jax.experimental.pallas import tpu_sc as plsc

N = 10000
E = 320000
D = 128
H = 8
OUT = 16
NC, NS, L = 2, 16, 16
C = 128
NCHUNK = E // C
NP_ = 10112
ROWS_PER_TILE = NP_ // NS
CH = 64

_BLK = 1000
_GRID = N // _BLK


def _proj_body(x_ref, w_ref, alr_ref, feat_ref, elr_ref):
    f = jnp.dot(x_ref[...], w_ref[...], preferred_element_type=jnp.float32)
    feat_ref[...] = f
    elr_ref[...] = jnp.dot(f, alr_ref[...], preferred_element_type=jnp.float32)


def _project(nfeat, w1, alr):
    return pl.pallas_call(
        _proj_body,
        grid=(_GRID,),
        in_specs=[
            pl.BlockSpec((_BLK, D), lambda i: (i, 0)),
            pl.BlockSpec((D, D), lambda i: (0, 0)),
            pl.BlockSpec((D, L), lambda i: (0, 0)),
        ],
        out_specs=[
            pl.BlockSpec((_BLK, D), lambda i: (i, 0)),
            pl.BlockSpec((_BLK, L), lambda i: (i, 0)),
        ],
        out_shape=[
            jax.ShapeDtypeStruct((N, D), jnp.float32),
            jax.ShapeDtypeStruct((N, L), jnp.float32),
        ],
    )(nfeat, w1, alr)


_mesh = plsc.VectorSubcoreMesh(
    core_axis_name="c", subcore_axis_name="s", num_cores=NC, num_subcores=NS)


@functools.partial(
    pl.kernel,
    out_type=(
        jax.ShapeDtypeStruct((NC, NP_, D), jnp.float32),
        jax.ShapeDtypeStruct((NP_, L), jnp.float32),
    ),
    mesh=_mesh,
    scratch_types=[
        pltpu.VMEM_SHARED((NP_, D), jnp.float32),
        pltpu.VMEM_SHARED((NP_, L), jnp.float32),
        pltpu.VMEM_SHARED((N, L), jnp.float32),
        pltpu.VMEM((2, CH), jnp.int32),
        pltpu.VMEM((2, CH), jnp.int32),
        pltpu.VMEM((2, CH, L), jnp.float32),
        pltpu.VMEM((2, CH, L), jnp.float32),
        pltpu.VMEM((CH, D), jnp.float32),
        pltpu.VMEM((CH, D), jnp.float32),
        pltpu.VMEM((CH, L), jnp.float32),
        pltpu.VMEM((CH, L), jnp.float32),
        pltpu.SemaphoreType.DMA,
        pltpu.SemaphoreType.DMA,
        pltpu.SemaphoreType.DMA,
        pltpu.SemaphoreType.DMA,
        pltpu.SemaphoreType.DMA,
        pltpu.SemaphoreType.DMA,
    ],
    compiler_params=pltpu.CompilerParams(needs_layout_passes=False,
                                         use_tc_tiling_on_sc=False),
)
def _edge_kernel(eidx_hbm, elr_hbm, feat_hbm, efeat_hbm,
                 acc_out, dd_out,
                 acc_sh, dd_sh, elr_sh,
                 idxA, idxB, ebA, ebB, featA, featB, wscA, wscB,
                 semA0, semA1, semA2, semB0, semB1, semB2):
    cid = lax.axis_index("c")
    sid = lax.axis_index("s")
    lanes = lax.iota(jnp.int32, L)
    lane_lt8 = lanes < 8
    lane8_one = jnp.where(lanes == 8, 1.0, 0.0).astype(jnp.float32)

    def _zrow(i, carry):
        for j in range(D // L):
            featA[i, pl.ds(j * L, L)] = jnp.zeros((L,), jnp.float32)
        wscA[i, :] = jnp.zeros((L,), jnp.float32)
        return carry
    lax.fori_loop(0, CH, _zrow, 0)

    base = sid * ROWS_PER_TILE
    for k in range(0, ROWS_PER_TILE, CH):
        cnt = min(CH, ROWS_PER_TILE - k)
        pltpu.sync_copy(featA.at[pl.ds(0, cnt)], acc_sh.at[pl.ds(base + k, cnt)])
        pltpu.sync_copy(wscA.at[pl.ds(0, cnt)], dd_sh.at[pl.ds(base + k, cnt)])

    @pl.when(cid == 0)
    def _stage():
        sbase = jnp.minimum(sid * 640, N - 640)
        pltpu.sync_copy(elr_hbm.at[pl.ds(sbase, 640)],
                        elr_sh.at[pl.ds(sbase, 640)])

    plsc.subcore_barrier()

    nloops = (NCHUNK + NS - 1) // NS

    @pl.when(cid == 0)
    def _attention_core():
        def _compute(eb2, featb, wsc):
            def _erow(i, c3):
                rot = eb2[1, i, :].at[(lanes + 8) % L].get(
                    mode="promise_in_bounds")
                x = eb2[0, i, :] + rot
                w = jnp.exp(jnp.maximum(x, 0.2 * x))
                wsc[i, :] = jnp.where(lane_lt8, w, lane8_one)
                for h in range(H):
                    wh = w.at[jnp.full((L,), h, jnp.int32)].get(
                        mode="promise_in_bounds")
                    featb[i, pl.ds(h * L, L)] = featb[i, pl.ds(h * L, L)] * wh
                return c3
            lax.fori_loop(0, CH, _erow, 0)

        def _chunk(j, carry):
            ck = sid + j * NS

            @pl.when(ck < NCHUNK)
            def _():
                eb = ck * C
                pltpu.sync_copy(eidx_hbm.at[:, pl.ds(eb, CH)], idxA)
                gA2 = pltpu.async_copy(feat_hbm.at[idxA.at[0]], featA, semA2)
                gA0 = pltpu.async_copy(elr_sh.at[idxA.at[0]], ebA.at[0], semA0)
                gA1 = pltpu.async_copy(elr_sh.at[idxA.at[1]], ebA.at[1], semA1)
                pltpu.sync_copy(eidx_hbm.at[:, pl.ds(eb + CH, CH)], idxB)
                gB2 = pltpu.async_copy(feat_hbm.at[idxB.at[0]], featB, semB2)
                gB0 = pltpu.async_copy(elr_sh.at[idxB.at[0]], ebB.at[0], semB0)
                gB1 = pltpu.async_copy(elr_sh.at[idxB.at[1]], ebB.at[1], semB1)
                gA0.wait()
                gA1.wait()
                gA2.wait()
                _compute(ebA, featA, wscA)
                pltpu.sync_copy(featA, acc_sh.at[idxA.at[1]], add=True)
                pltpu.sync_copy(wscA, dd_sh.at[idxA.at[1]], add=True)
                gB0.wait()
                gB1.wait()
                gB2.wait()
                _compute(ebB, featB, wscB)
                pltpu.sync_copy(featB, acc_sh.at[idxB.at[1]], add=True)
                pltpu.sync_copy(wscB, dd_sh.at[idxB.at[1]], add=True)
            return carry
        lax.fori_loop(0, nloops, _chunk, 0)

    @pl.when(cid == 1)
    def _esum_core():
        def _chunk(j, carry):
            ck = sid + j * NS

            @pl.when(ck < NCHUNK)
            def _():
                eb = ck * C
                pltpu.sync_copy(eidx_hbm.at[:, pl.ds(eb, CH)], idxA)
                gA = pltpu.async_copy(efeat_hbm.at[pl.ds(eb, CH)], featA, semA2)
                pltpu.sync_copy(eidx_hbm.at[:, pl.ds(eb + CH, CH)], idxB)
                gB = pltpu.async_copy(efeat_hbm.at[pl.ds(eb + CH, CH)], featB,
                                      semB2)
                gA.wait()
                pltpu.sync_copy(featA, acc_sh.at[idxA.at[1]], add=True)
                gB.wait()
                pltpu.sync_copy(featB, acc_sh.at[idxB.at[1]], add=True)
            return carry
        lax.fori_loop(0, nloops, _chunk, 0)

    plsc.subcore_barrier()

    pltpu.sync_copy(acc_sh.at[pl.ds(base, ROWS_PER_TILE)],
                    acc_out.at[cid, pl.ds(base, ROWS_PER_TILE)])

    @pl.when(cid == 0)
    def _drain_dd():
        pltpu.sync_copy(dd_sh.at[pl.ds(base, ROWS_PER_TILE)],
                        dd_out.at[pl.ds(base, ROWS_PER_TILE)])


def _comb_body(num_ref, esum_ref, dd_ref, deg_sel_ref, feat_ref, ew_ref, e16_ref,
               gb_ref, eb_ref, o_ref):
    dd = dd_ref[...]
    denom = jnp.dot(dd, e16_ref[...], preferred_element_type=jnp.float32)
    deg = jnp.dot(dd, deg_sel_ref[...], preferred_element_type=jnp.float32)
    rst = num_ref[...] / jnp.maximum(denom, 1e-9)
    he = (jnp.dot(esum_ref[...], ew_ref[...], preferred_element_type=jnp.float32)
          / jnp.maximum(deg, 1.0)) + eb_ref[...] * jnp.minimum(deg, 1.0)
    o_ref[...] = rst + gb_ref[...] + feat_ref[...] / (deg + 1.0) + he


def _combine(num, esum, dn, deg, feat, ewT, e8, gb, eb):
    return pl.pallas_call(
        _comb_body,
        grid=(_GRID,),
        in_specs=[
            pl.BlockSpec((_BLK, D), lambda i: (i, 0)),
            pl.BlockSpec((_BLK, D), lambda i: (i, 0)),
            pl.BlockSpec((_BLK, L), lambda i: (i, 0)),
            pl.BlockSpec((L, 1), lambda i: (0, 0)),
            pl.BlockSpec((_BLK, D), lambda i: (i, 0)),
            pl.BlockSpec((D, D), lambda i: (0, 0)),
            pl.BlockSpec((L, D), lambda i: (0, 0)),
            pl.BlockSpec((1, D), lambda i: (0, 0)),
            pl.BlockSpec((1, D), lambda i: (0, 0)),
        ],
        out_specs=pl.BlockSpec((_BLK, D), lambda i: (i, 0)),
        out_shape=jax.ShapeDtypeStruct((N, D), jnp.float32),
    )(num, esum, dn, deg, feat, ewT, e8, gb, eb)


def kernel(nfeat, edge_index, efeat, fc_w, attn_l, attn_r, gat_bias, edge_w,
           edge_b):
    cols = jnp.arange(D)
    head = cols // OUT
    alr = (jnp.zeros((D, L), jnp.float32)
           .at[cols, head].set(attn_l.reshape(-1))
           .at[cols, 8 + head].set(attn_r.reshape(-1)))
    feat, elrtab = _project(nfeat, fc_w.T, alr)

    acc, dd = _edge_kernel(edge_index, elrtab, feat, efeat)

    e16 = (jnp.arange(L)[:, None] == head[None, :]).astype(jnp.float32)
    dsel = (jnp.arange(L)[:, None] == 8).astype(jnp.float32)
    out = _combine(acc[0, :N], acc[1, :N], dd[:N], dsel, feat, edge_w.T, e16,
                   gat_bias.reshape(1, D), edge_b.reshape(1, D))
    return out


# speedup vs baseline: 69.3452x; 1.5925x over previous
import functools

import jax
import jax.numpy as jnp
from jax import lax
from jax.experimental import pallas as pl
from jax.experimental.pallas import tpu as pltpu
from jax.experimental.pallas import tpu_sc as plsc

N = 10000
E = 320000
D = 128
H = 8
OUT = 16
NC, NS, L = 2, 16, 16
C = 128
NCHUNK = E // C
NP_ = 10112
ROWS_PER_TILE = NP_ // NS
CH = 64

_BLK = 1000
_GRID = N // _BLK


def _proj_body(x_ref, w_ref, alr_ref, feat_ref, elr_ref):
    f = jnp.dot(x_ref[...], w_ref[...], preferred_element_type=jnp.float32)
    feat_ref[...] = f
    elr_ref[...] = jnp.dot(f, alr_ref[...], preferred_element_type=jnp.float32)


def _project(nfeat, w1, alr):
    return pl.pallas_call(
        _proj_body,
        grid=(_GRID,),
        in_specs=[
            pl.BlockSpec((_BLK, D), lambda i: (i, 0)),
            pl.BlockSpec((D, D), lambda i: (0, 0)),
            pl.BlockSpec((D, L), lambda i: (0, 0)),
        ],
        out_specs=[
            pl.BlockSpec((_BLK, D), lambda i: (i, 0)),
            pl.BlockSpec((_BLK, L), lambda i: (i, 0)),
        ],
        out_shape=[
            jax.ShapeDtypeStruct((N, D), jnp.float32),
            jax.ShapeDtypeStruct((N, L), jnp.float32),
        ],
    )(nfeat, w1, alr)


_mesh = plsc.VectorSubcoreMesh(
    core_axis_name="c", subcore_axis_name="s", num_cores=NC, num_subcores=NS)


@functools.partial(
    pl.kernel,
    out_type=(
        jax.ShapeDtypeStruct((NC, NP_, D), jnp.float32),
        jax.ShapeDtypeStruct((NP_, L), jnp.float32),
    ),
    mesh=_mesh,
    scratch_types=[
        pltpu.VMEM_SHARED((NP_, D), jnp.float32),
        pltpu.VMEM_SHARED((NP_, L), jnp.float32),
        pltpu.VMEM_SHARED((N, L), jnp.float32),
        pltpu.VMEM((2, CH), jnp.int32),
        pltpu.VMEM((2, CH), jnp.int32),
        pltpu.VMEM((2, CH, L), jnp.float32),
        pltpu.VMEM((2, CH, L), jnp.float32),
        pltpu.VMEM((CH, D), jnp.float32),
        pltpu.VMEM((CH, D), jnp.float32),
        pltpu.VMEM((CH, L), jnp.float32),
        pltpu.VMEM((CH, L), jnp.float32),
        pltpu.SemaphoreType.DMA,
        pltpu.SemaphoreType.DMA,
        pltpu.SemaphoreType.DMA,
        pltpu.SemaphoreType.DMA,
        pltpu.SemaphoreType.DMA,
        pltpu.SemaphoreType.DMA,
    ],
    compiler_params=pltpu.CompilerParams(needs_layout_passes=False,
                                         use_tc_tiling_on_sc=False),
)
def _edge_kernel(eidx_hbm, elr_hbm, feat_hbm, efeat_hbm,
                 acc_out, dd_out,
                 acc_sh, dd_sh, elr_sh,
                 idxA, idxB, ebA, ebB, featA, featB, wscA, wscB,
                 semA0, semA1, semA2, semB0, semB1, semB2):
    cid = lax.axis_index("c")
    sid = lax.axis_index("s")
    lanes = lax.iota(jnp.int32, L)
    lane_lt8 = lanes < 8
    lane8_one = jnp.where(lanes == 8, 1.0, 0.0).astype(jnp.float32)

    def _zrow(i, carry):
        for j in range(D // L):
            featA[i, pl.ds(j * L, L)] = jnp.zeros((L,), jnp.float32)
        wscA[i, :] = jnp.zeros((L,), jnp.float32)
        return carry
    lax.fori_loop(0, CH, _zrow, 0)

    base = sid * ROWS_PER_TILE
    for k in range(0, ROWS_PER_TILE, CH):
        cnt = min(CH, ROWS_PER_TILE - k)
        pltpu.sync_copy(featA.at[pl.ds(0, cnt)], acc_sh.at[pl.ds(base + k, cnt)])
        pltpu.sync_copy(wscA.at[pl.ds(0, cnt)], dd_sh.at[pl.ds(base + k, cnt)])

    @pl.when(cid == 0)
    def _stage():
        sbase = jnp.minimum(sid * 640, N - 640)
        pltpu.sync_copy(elr_hbm.at[pl.ds(sbase, 640)],
                        elr_sh.at[pl.ds(sbase, 640)])

    plsc.subcore_barrier()

    nloops = (NCHUNK + NS - 1) // NS

    @pl.when(cid == 0)
    def _attention_core():
        def _compute(eb2, featb, wsc):
            @plsc.parallel_loop(0, CH, unroll=4)
            def _erow(i):
                rot = eb2[1, i, :].at[(lanes + 8) % L].get(
                    mode="promise_in_bounds")
                x = eb2[0, i, :] + rot
                w = jnp.exp(jnp.maximum(x, 0.2 * x))
                wsc[i, :] = jnp.where(lane_lt8, w, lane8_one)
                for h in range(H):
                    wh = w.at[jnp.full((L,), h, jnp.int32)].get(
                        mode="promise_in_bounds")
                    featb[i, pl.ds(h * L, L)] = featb[i, pl.ds(h * L, L)] * wh

        def _chunk(j, carry):
            ck = sid + j * NS

            @pl.when(ck < NCHUNK)
            def _():
                eb = ck * C
                pltpu.sync_copy(eidx_hbm.at[:, pl.ds(eb, CH)], idxA)
                gA2 = pltpu.async_copy(feat_hbm.at[idxA.at[0]], featA, semA2)
                gA0 = pltpu.async_copy(elr_sh.at[idxA.at[0]], ebA.at[0], semA0)
                gA1 = pltpu.async_copy(elr_sh.at[idxA.at[1]], ebA.at[1], semA1)
                pltpu.sync_copy(eidx_hbm.at[:, pl.ds(eb + CH, CH)], idxB)
                gB2 = pltpu.async_copy(feat_hbm.at[idxB.at[0]], featB, semB2)
                gB0 = pltpu.async_copy(elr_sh.at[idxB.at[0]], ebB.at[0], semB0)
                gB1 = pltpu.async_copy(elr_sh.at[idxB.at[1]], ebB.at[1], semB1)
                gA0.wait()
                gA1.wait()
                gA2.wait()
                _compute(ebA, featA, wscA)
                pltpu.sync_copy(featA, acc_sh.at[idxA.at[1]], add=True)
                pltpu.sync_copy(wscA, dd_sh.at[idxA.at[1]], add=True)
                gB0.wait()
                gB1.wait()
                gB2.wait()
                _compute(ebB, featB, wscB)
                pltpu.sync_copy(featB, acc_sh.at[idxB.at[1]], add=True)
                pltpu.sync_copy(wscB, dd_sh.at[idxB.at[1]], add=True)
            return carry
        lax.fori_loop(0, nloops, _chunk, 0)

    @pl.when(cid == 1)
    def _esum_core():
        def _chunk(j, carry):
            ck = sid + j * NS

            @pl.when(ck < NCHUNK)
            def _():
                eb = ck * C
                pltpu.sync_copy(eidx_hbm.at[:, pl.ds(eb, CH)], idxA)
                gA = pltpu.async_copy(efeat_hbm.at[pl.ds(eb, CH)], featA, semA2)
                pltpu.sync_copy(eidx_hbm.at[:, pl.ds(eb + CH, CH)], idxB)
                gB = pltpu.async_copy(efeat_hbm.at[pl.ds(eb + CH, CH)], featB,
                                      semB2)
                gA.wait()
                pltpu.sync_copy(featA, acc_sh.at[idxA.at[1]], add=True)
                gB.wait()
                pltpu.sync_copy(featB, acc_sh.at[idxB.at[1]], add=True)
            return carry
        lax.fori_loop(0, nloops, _chunk, 0)

    plsc.subcore_barrier()

    pltpu.sync_copy(acc_sh.at[pl.ds(base, ROWS_PER_TILE)],
                    acc_out.at[cid, pl.ds(base, ROWS_PER_TILE)])

    @pl.when(cid == 0)
    def _drain_dd():
        pltpu.sync_copy(dd_sh.at[pl.ds(base, ROWS_PER_TILE)],
                        dd_out.at[pl.ds(base, ROWS_PER_TILE)])


def _comb_body(num_ref, esum_ref, dd_ref, deg_sel_ref, feat_ref, ew_ref, e16_ref,
               gb_ref, eb_ref, o_ref):
    dd = dd_ref[...]
    denom = jnp.dot(dd, e16_ref[...], preferred_element_type=jnp.float32)
    deg = jnp.dot(dd, deg_sel_ref[...], preferred_element_type=jnp.float32)
    rst = num_ref[...] / jnp.maximum(denom, 1e-9)
    he = (jnp.dot(esum_ref[...], ew_ref[...], preferred_element_type=jnp.float32)
          / jnp.maximum(deg, 1.0)) + eb_ref[...] * jnp.minimum(deg, 1.0)
    o_ref[...] = rst + gb_ref[...] + feat_ref[...] / (deg + 1.0) + he


def _combine(num, esum, dn, deg, feat, ewT, e8, gb, eb):
    return pl.pallas_call(
        _comb_body,
        grid=(_GRID,),
        in_specs=[
            pl.BlockSpec((_BLK, D), lambda i: (i, 0)),
            pl.BlockSpec((_BLK, D), lambda i: (i, 0)),
            pl.BlockSpec((_BLK, L), lambda i: (i, 0)),
            pl.BlockSpec((L, 1), lambda i: (0, 0)),
            pl.BlockSpec((_BLK, D), lambda i: (i, 0)),
            pl.BlockSpec((D, D), lambda i: (0, 0)),
            pl.BlockSpec((L, D), lambda i: (0, 0)),
            pl.BlockSpec((1, D), lambda i: (0, 0)),
            pl.BlockSpec((1, D), lambda i: (0, 0)),
        ],
        out_specs=pl.BlockSpec((_BLK, D), lambda i: (i, 0)),
        out_shape=jax.ShapeDtypeStruct((N, D), jnp.float32),
    )(num, esum, dn, deg, feat, ewT, e8, gb, eb)


def kernel(nfeat, edge_index, efeat, fc_w, attn_l, attn_r, gat_bias, edge_w,
           edge_b):
    cols = jnp.arange(D)
    head = cols // OUT
    alr = (jnp.zeros((D, L), jnp.float32)
           .at[cols, head].set(attn_l.reshape(-1))
           .at[cols, 8 + head].set(attn_r.reshape(-1)))
    feat, elrtab = _project(nfeat, fc_w.T, alr)

    acc, dd = _edge_kernel(edge_index, elrtab, feat, efeat)

    e16 = (jnp.arange(L)[:, None] == head[None, :]).astype(jnp.float32)
    dsel = (jnp.arange(L)[:, None] == 8).astype(jnp.float32)
    out = _combine(acc[0, :N], acc[1, :N], dd[:N], dsel, feat, edge_w.T, e16,
                   gat_bias.reshape(1, D), edge_b.reshape(1, D))
    return out
